# baseline (speedup 1.0000x reference)
import functools

import jax
import jax.numpy as jnp
from jax import lax
from jax.experimental import pallas as pl
from jax.experimental.pallas import tpu as pltpu
from jax.experimental.pallas import tpu_sc as plsc

_RPW = 64


def _sc_body(flat_hbm, out_hbm, win1, win128, q_sh, sem):
    c = lax.axis_index("c")
    s = lax.axis_index("s")
    wid = s * 2 + c

    p = s // 2
    h = s % 2
    src0 = pl.multiple_of(16 * p + 32768 * h, 16)
    pltpu.sync_copy(flat_hbm.at[pl.ds(src0, 32768)], win1)

    def repack(u, carry):
        for l8 in range(8):
            win128[u, pl.ds(16 * l8, 16)] = win1[pl.ds(128 * u + 16 * l8, 16)]
        return carry

    lax.fori_loop(0, 256, repack, 0)
    pltpu.sync_copy(win128, q_sh.at[p, pl.ds(256 * h, 256), :])
    plsc.subcore_barrier()

    base = _RPW * wid
    for chunk in range(0, _RPW, 16):
        copies = [
            pltpu.make_async_copy(
                q_sh.at[7 - (r % 8), pl.ds(255 - 8 * wid - (r // 8), 256), :],
                out_hbm.at[base + r],
                sem,
            )
            for r in range(chunk, chunk + 16)
        ]
        for cp in copies:
            cp.start()
        for cp in copies:
            cp.wait()


def _sc_write(flat1d):
    mesh = plsc.VectorSubcoreMesh(core_axis_name="c", subcore_axis_name="s")
    run = functools.partial(
        pl.kernel,
        mesh=mesh,
        out_type=jax.ShapeDtypeStruct((2048, 256, 128), jnp.float32),
        scratch_types=[
            pltpu.VMEM((32768,), jnp.float32),
            pltpu.VMEM((256, 128), jnp.float32),
            pltpu.VMEM_SHARED((8, 512, 128), jnp.float32),
            pltpu.SemaphoreType.DMA,
        ],
    )(_sc_body)
    return run(flat1d)


def _tc_body(f_ref, prev_ref, out_ref, q, sem):
    del prev_ref
    f = f_ref[...]
    for p in range(8):
        if p == 0:
            q[0] = f[0:512, :]
        else:
            q[p] = jnp.concatenate([f[0:512, 16 * p:], f[1:513, : 16 * p]], axis=1)

    copies = [
        pltpu.make_async_copy(
            q.at[:, pl.ds(255 - b, 256), :],
            out_ref.at[pl.ds(8 * b, 8), :, :],
            sem,
        )
        for b in range(8)
    ]
    for cp in copies:
        cp.start()
    for cp in copies:
        cp.wait()


def _tc_write(f2d, out_sc):
    return pl.pallas_call(
        _tc_body,
        in_specs=[
            pl.BlockSpec(memory_space=pltpu.MemorySpace.VMEM),
            pl.BlockSpec(memory_space=pl.ANY),
        ],
        out_specs=pl.BlockSpec(memory_space=pl.ANY),
        out_shape=jax.ShapeDtypeStruct((2048, 256, 128), jnp.float32),
        scratch_shapes=[
            pltpu.VMEM((8, 512, 128), jnp.float32),
            pltpu.SemaphoreType.DMA,
        ],
        input_output_aliases={1: 0},
    )(f2d, out_sc)


def kernel(inputs, rel_embeddings):
    del inputs
    flat = jnp.flip(rel_embeddings, axis=0).reshape(-1)
    flat1 = jnp.concatenate([flat, jnp.zeros((144,), flat.dtype)])
    out = _sc_write(flat1[:65536])
    out = _tc_write(flat1.reshape(513, 128), out)
    return out.reshape(2048, 2048, 16)


# speedup vs baseline: 1.0142x; 1.0142x over previous
import functools

import jax
import jax.numpy as jnp
from jax import lax
from jax.experimental import pallas as pl
from jax.experimental.pallas import tpu as pltpu
from jax.experimental.pallas import tpu_sc as plsc

_RPW = 64


def _build_body(f_ref, q_ref):
    f = f_ref[...]
    for p in range(8):
        if p == 0:
            q_ref[0] = f[0:512, :]
        else:
            q_ref[p] = jnp.concatenate(
                [f[0:512, 16 * p:], f[1:513, : 16 * p]], axis=1
            )


def _build_phase_table(f2d):
    return pl.pallas_call(
        _build_body,
        in_specs=[pl.BlockSpec(memory_space=pltpu.MemorySpace.VMEM)],
        out_specs=pl.BlockSpec(memory_space=pltpu.MemorySpace.VMEM),
        out_shape=jax.ShapeDtypeStruct((8, 512, 128), jnp.float32),
    )(f2d)


def _sc_body(q_hbm, out_hbm, q_sh, sem):
    c = lax.axis_index("c")
    s = lax.axis_index("s")
    wid = s * 2 + c

    @pl.when(s == 0)
    def _():
        pltpu.sync_copy(q_hbm, q_sh)

    plsc.subcore_barrier()

    base = _RPW * wid
    for chunk in range(0, _RPW, 16):
        copies = [
            pltpu.make_async_copy(
                q_sh.at[7 - (r % 8), pl.ds(255 - 8 * wid - (r // 8), 256), :],
                out_hbm.at[base + r],
                sem,
            )
            for r in range(chunk, chunk + 16)
        ]
        for cp in copies:
            cp.start()
        for cp in copies:
            cp.wait()


def _sc_write(q):
    mesh = plsc.VectorSubcoreMesh(core_axis_name="c", subcore_axis_name="s")
    run = functools.partial(
        pl.kernel,
        mesh=mesh,
        out_type=jax.ShapeDtypeStruct((2048, 256, 128), jnp.float32),
        scratch_types=[
            pltpu.VMEM_SHARED((8, 512, 128), jnp.float32),
            pltpu.SemaphoreType.DMA,
        ],
    )(_sc_body)
    return run(q)


def kernel(inputs, rel_embeddings):
    del inputs
    flat = jnp.flip(rel_embeddings, axis=0).reshape(-1)
    f2d = jnp.concatenate([flat, jnp.zeros((144,), flat.dtype)]).reshape(513, 128)
    q = _build_phase_table(f2d)
    out = _sc_write(q)
    return out.reshape(2048, 2048, 16)
